# Initial kernel scaffold; baseline (speedup 1.0000x reference)
import jax
import jax.numpy as jnp
from jax.experimental import pallas as pl


def kernel(CB_coords, v_bins):
    raise NotImplementedError("write your pallas kernel here")


# speedup vs baseline: 121.2227x; 121.2227x over previous
import jax
import jax.numpy as jnp
from jax.experimental import pallas as pl
from jax.experimental.pallas import tpu as pltpu

_NBINS = 38
_IBLK = 32


def _onehot_kernel(at_ref, b_ref, aux_ref, out_ref):
    d2 = None
    for c in range(3):
        a = at_ref[0, 0, c : c + 1, :]
        b = b_ref[0, :, c : c + 1]
        diff = b - a
        d2 = diff * diff if d2 is None else d2 + diff * diff
    d = jnp.sqrt(d2)
    start = aux_ref[1:2, 0:1]
    inv_step = aux_ref[1:2, 1:2]
    bidx = jnp.clip(jnp.floor((d - start) * inv_step), 0.0, _NBINS - 1.0)
    iota = aux_ref[0:1, 0:_NBINS]
    for i in range(_IBLK):
        col = bidx[:, i : i + 1]
        out_ref[0, i] = jnp.where(col == iota, 1.0, 0.0)


def kernel(CB_coords, v_bins):
    nbatch, n, _ = CB_coords.shape
    coords_t = jnp.transpose(CB_coords, (0, 2, 1))
    coords_t = jnp.pad(coords_t, ((0, 0), (0, 5), (0, 0)))
    coords_t = jnp.transpose(
        coords_t.reshape(nbatch, 8, n // _IBLK, _IBLK), (0, 2, 1, 3)
    )
    coords_p = jnp.pad(CB_coords, ((0, 0), (0, 0), (0, 5)))
    start = v_bins[0]
    inv_step = 1.0 / (v_bins[1] - v_bins[0])
    aux = jnp.zeros((8, 128), jnp.float32)
    aux = aux.at[0, :_NBINS].set(jnp.arange(_NBINS, dtype=jnp.float32))
    aux = aux.at[1, 0].set(start)
    aux = aux.at[1, 1].set(inv_step)

    grid = (nbatch, n // _IBLK)
    out = pl.pallas_call(
        _onehot_kernel,
        grid=grid,
        in_specs=[
            pl.BlockSpec((1, 1, 8, _IBLK), lambda b, i: (b, i, 0, 0)),
            pl.BlockSpec((1, n, 8), lambda b, i: (b, 0, 0)),
            pl.BlockSpec((8, 128), lambda b, i: (0, 0)),
        ],
        out_specs=pl.BlockSpec((1, _IBLK, n, _NBINS), lambda b, i: (b, i, 0, 0)),
        out_shape=jax.ShapeDtypeStruct((nbatch, n, n, _NBINS), jnp.float32),
    )(coords_t, coords_p, aux)
    return out


# speedup vs baseline: 142.8001x; 1.1780x over previous
import jax
import jax.numpy as jnp
from jax.experimental import pallas as pl
from jax.experimental.pallas import tpu as pltpu

_NBINS = 38
_IBLK = 64
_JCHUNK = 128
_FLATC = _JCHUNK * _NBINS


def _onehot_kernel(at_ref, bt_ref, aux_ref, w_ref, kflat_ref, out_ref):
    n = bt_ref.shape[2]
    d2 = None
    for c in range(3):
        a = at_ref[0, 0, :, c : c + 1]
        b = bt_ref[0, c : c + 1, :]
        diff = a - b
        d2 = diff * diff if d2 is None else d2 + diff * diff
    d = jnp.sqrt(d2)
    start = aux_ref[1:2, 0:1]
    inv_step = aux_ref[1:2, 1:2]
    bidx = jnp.clip(jnp.floor((d - start) * inv_step), 0.0, _NBINS - 1.0)
    bidx16 = bidx.astype(jnp.bfloat16)
    kflat = kflat_ref[0:1, :]
    for c in range(n // _JCHUNK):
        bc = bidx16[:, c * _JCHUNK : (c + 1) * _JCHUNK]
        m = jnp.dot(bc, w_ref[:, :], preferred_element_type=jnp.float32)
        out_ref[0, :, c * _FLATC : (c + 1) * _FLATC] = jnp.where(
            m == kflat, 1.0, 0.0
        )


def kernel(CB_coords, v_bins):
    nbatch, n, _ = CB_coords.shape
    nblk = n // _IBLK
    coords_p = jnp.pad(CB_coords, ((0, 0), (0, 0), (0, 5)))
    coords_rows = coords_p.reshape(nbatch, nblk, _IBLK, 8)
    coords_t = jnp.pad(
        jnp.transpose(CB_coords, (0, 2, 1)), ((0, 0), (0, 5), (0, 0))
    )
    aux = jnp.zeros((8, 128), jnp.float32)
    aux = aux.at[1, 0].set(v_bins[0])
    aux = aux.at[1, 1].set(1.0 / (v_bins[1] - v_bins[0]))
    p = jnp.arange(_FLATC, dtype=jnp.int32)
    w = (p[None, :] // _NBINS == jnp.arange(_JCHUNK, dtype=jnp.int32)[:, None])
    w = w.astype(jnp.bfloat16)
    kflat = (p % _NBINS).astype(jnp.float32)[None, :]

    grid = (nbatch, nblk)
    out = pl.pallas_call(
        _onehot_kernel,
        grid=grid,
        in_specs=[
            pl.BlockSpec((1, 1, _IBLK, 8), lambda b, i: (b, i, 0, 0)),
            pl.BlockSpec((1, 8, n), lambda b, i: (b, 0, 0)),
            pl.BlockSpec((8, 128), lambda b, i: (0, 0)),
            pl.BlockSpec((_JCHUNK, _FLATC), lambda b, i: (0, 0)),
            pl.BlockSpec((1, _FLATC), lambda b, i: (0, 0)),
        ],
        out_specs=pl.BlockSpec((1, _IBLK, n * _NBINS), lambda b, i: (b, i, 0)),
        out_shape=jax.ShapeDtypeStruct((nbatch, n, n * _NBINS), jnp.float32),
    )(coords_rows, coords_t, aux, w, kflat)
    return out.reshape(nbatch, n, n, _NBINS)
